# baseline (speedup 1.0000x reference)
import functools

import jax
import jax.numpy as jnp
from jax import lax
from jax.experimental import pallas as pl
from jax.experimental.pallas import tpu as pltpu
from jax.experimental.pallas import tpu_sc as plsc

N_NODES = 100000
N_EDGES = 3200000
D_FEAT = 128
D_EDGE = 16
N_GRAPHS = 64
D_U = 64
OUT_DIM = 128

N_SC = 2048000

NC, NS, L = 2, 16, 16
NW = NC * NS
CE = 1024
NCH = N_SC // CE
CH_PER_W = 2 * (-(-NCH // (2 * NW)))
BPK_WORDS = N_NODES // 4

BN = 2000
NB = N_NODES // BN
BE = 16000
EB = N_EDGES // BE
SC_EB = N_SC // BE
TC_EB = EB - SC_EB


def _sc_edge_body(eidx_hbm, eattr_hbm, bpk_hbm, out_hbm,
                  bpk_v, idx_v0, idx_v1, attr_v0, attr_v1, acc_v,
                  isem0, isem1, asem0, asem1):
    c = lax.axis_index("c")
    s = lax.axis_index("s")
    wid = s * NC + c
    idx_b = (idx_v0, idx_v1)
    attr_b = (attr_v0, attr_v1)
    isem = (isem0, isem1)
    asem = (asem0, asem1)

    pltpu.sync_copy(bpk_hbm, bpk_v)

    def zrow(r, carry):
        acc_v[pl.ds(r * L, L)] = jnp.zeros((L,), jnp.float32)
        return carry
    lax.fori_loop(0, N_GRAPHS, zrow, 0)

    lane = lax.iota(jnp.int32, L)

    def issue(jj, b):
        cid = wid + jj * NW

        @pl.when(cid < NCH)
        def _():
            base = cid * CE
            pltpu.make_async_copy(
                eidx_hbm.at[0, pl.ds(base, CE)], idx_b[b], isem[b]).start()
            pltpu.make_async_copy(
                eattr_hbm.at[pl.ds(base * D_EDGE, CE * D_EDGE)],
                attr_b[b], asem[b]).start()

    def consume(jj, b):
        cid = wid + jj * NW

        @pl.when(cid < NCH)
        def _():
            base = cid * CE
            pltpu.make_async_copy(
                eidx_hbm.at[0, pl.ds(base, CE)], idx_b[b], isem[b]).wait()
            pltpu.make_async_copy(
                eattr_hbm.at[pl.ds(base * D_EDGE, CE * D_EDGE)],
                attr_b[b], asem[b]).wait()

            def group(e16, carry2):
                idx16 = idx_b[b][pl.ds(e16 * L, L)]
                w = plsc.load_gather(
                    bpk_v, [lax.shift_right_logical(idx16, 2)])
                seg16 = lax.shift_right_logical(w, (idx16 & 3) * 8) & 255
                addr16 = seg16 * D_EDGE
                for k in range(L):
                    row = attr_b[b][pl.ds((e16 * L + k) * D_EDGE, D_EDGE)]
                    a = jnp.take(addr16, jnp.full((L,), k, jnp.int32))
                    plsc.addupdate_scatter(acc_v, [a + lane], row)
                return carry2
            lax.fori_loop(0, CE // L, group, 0)

    issue(0, 0)

    def outer(j2, carry):
        jj = j2 * 2
        issue(jj + 1, 1)
        consume(jj, 0)
        issue(jj + 2, 0)
        consume(jj + 1, 1)
        return carry
    lax.fori_loop(0, CH_PER_W // 2, outer, 0)

    pltpu.sync_copy(acc_v, out_hbm.at[wid])


_sc_edge = functools.partial(
    pl.kernel,
    out_type=jax.ShapeDtypeStruct((NW, N_GRAPHS * D_EDGE), jnp.float32),
    mesh=plsc.VectorSubcoreMesh(core_axis_name="c", subcore_axis_name="s"),
    compiler_params=pltpu.CompilerParams(needs_layout_passes=False),
    scratch_types=[
        pltpu.VMEM((BPK_WORDS,), jnp.int32),
        pltpu.VMEM((CE,), jnp.int32),
        pltpu.VMEM((CE,), jnp.int32),
        pltpu.VMEM((CE * D_EDGE,), jnp.float32),
        pltpu.VMEM((CE * D_EDGE,), jnp.float32),
        pltpu.VMEM((N_GRAPHS * D_EDGE,), jnp.float32),
        pltpu.SemaphoreType.DMA,
        pltpu.SemaphoreType.DMA,
        pltpu.SemaphoreType.DMA,
        pltpu.SemaphoreType.DMA,
    ],
)(_sc_edge_body)


def _tc_agg_body(batch_ref, x_ref, eidx_ref, eattr_ref,
                 node_ref, edge_ref, starts_ref, ends_ref):
    t = pl.program_id(0)

    @pl.when(t == 0)
    def _init():
        node_ref[...] = jnp.zeros_like(node_ref)
        edge_ref[...] = jnp.zeros_like(edge_ref)
        starts_ref[...] = jnp.zeros_like(starts_ref)
        ends_ref[...] = jnp.zeros_like(ends_ref)

    @pl.when(t < NB)
    def _node():
        b_blk = batch_ref[0, 0, :].astype(jnp.int32)[None, :]
        g = lax.broadcasted_iota(jnp.int32, (N_GRAPHS, 1), 0)
        lt = (b_blk < g).astype(jnp.int32)
        le = (b_blk <= g).astype(jnp.int32)
        onehot_t = (le - lt).astype(jnp.float32)
        starts_ref[...] += jnp.sum(lt, axis=1, keepdims=True)
        ends_ref[...] += jnp.sum(le, axis=1, keepdims=True)
        node_ref[...] += jnp.dot(onehot_t, x_ref[...],
                                 preferred_element_type=jnp.float32)

    @pl.when(t >= NB)
    def _edge():
        idx = eidx_ref[0, 0, 0, :].astype(jnp.int32)[None, :]
        s = starts_ref[...]
        e = ends_ref[...]
        onehot_t = ((idx >= s) & (idx < e)).astype(jnp.float32)
        edge_ref[...] += jnp.dot(onehot_t, eattr_ref[...],
                                 preferred_element_type=jnp.float32)


def _tc_fin_body(node_ref, etc_ref, esc_ref, u_ref, W_ref, b_ref, out_ref):
    edge_agg = etc_ref[...] + jnp.sum(esc_ref[...], axis=0)
    inp = jnp.concatenate(
        [node_ref[...], edge_agg, u_ref[...]], axis=-1)
    out = jnp.dot(inp, W_ref[...], preferred_element_type=jnp.float32)
    out_ref[...] = jnp.maximum(out + b_ref[...], 0.0)


@jax.jit
def kernel(x, edge_index, edge_attr, u, batch, W, b):
    batch_i32 = batch.astype(jnp.int32)
    eidx = edge_index.astype(jnp.int32)

    bpk = lax.bitcast_convert_type(
        batch_i32.astype(jnp.uint8).reshape(BPK_WORDS, 4), jnp.int32)
    eattr_sc = lax.slice(edge_attr, (0, 0), (N_SC, D_EDGE)).reshape(-1)
    edge_part = _sc_edge(eidx, eattr_sc, bpk)
    edge_part = edge_part.reshape(NW, N_GRAPHS, D_EDGE)

    batch3 = batch_i32.reshape(NB, 1, BN)
    eidx4 = eidx.reshape(2, EB, 1, BE)
    b2 = b.reshape(1, OUT_DIM)

    node_agg, edge_tc = pl.pallas_call(
        _tc_agg_body,
        grid=(NB + TC_EB,),
        in_specs=[
            pl.BlockSpec((1, 1, BN), lambda t: (jnp.minimum(t, NB - 1), 0, 0)),
            pl.BlockSpec((BN, D_FEAT), lambda t: (jnp.minimum(t, NB - 1), 0)),
            pl.BlockSpec((1, 1, 1, BE),
                         lambda t: (0, SC_EB + jnp.clip(t - NB, 0, TC_EB - 1),
                                    0, 0)),
            pl.BlockSpec((BE, D_EDGE),
                         lambda t: (SC_EB + jnp.clip(t - NB, 0, TC_EB - 1),
                                    0)),
        ],
        out_specs=[
            pl.BlockSpec((N_GRAPHS, D_FEAT), lambda t: (0, 0)),
            pl.BlockSpec((N_GRAPHS, D_EDGE), lambda t: (0, 0)),
        ],
        out_shape=[
            jax.ShapeDtypeStruct((N_GRAPHS, D_FEAT), jnp.float32),
            jax.ShapeDtypeStruct((N_GRAPHS, D_EDGE), jnp.float32),
        ],
        scratch_shapes=[
            pltpu.VMEM((N_GRAPHS, 1), jnp.int32),
            pltpu.VMEM((N_GRAPHS, 1), jnp.int32),
        ],
    )(batch3, x, eidx4, edge_attr)

    out = pl.pallas_call(
        _tc_fin_body,
        grid=(1,),
        in_specs=[
            pl.BlockSpec((N_GRAPHS, D_FEAT), lambda t: (0, 0)),
            pl.BlockSpec((N_GRAPHS, D_EDGE), lambda t: (0, 0)),
            pl.BlockSpec((NW, N_GRAPHS, D_EDGE), lambda t: (0, 0, 0)),
            pl.BlockSpec((N_GRAPHS, D_U), lambda t: (0, 0)),
            pl.BlockSpec((D_FEAT + D_EDGE + D_U, OUT_DIM), lambda t: (0, 0)),
            pl.BlockSpec((1, OUT_DIM), lambda t: (0, 0)),
        ],
        out_specs=pl.BlockSpec((N_GRAPHS, OUT_DIM), lambda t: (0, 0)),
        out_shape=jax.ShapeDtypeStruct((N_GRAPHS, OUT_DIM), jnp.float32),
    )(node_agg, edge_tc, edge_part, u, W, b2)
    return out


# speedup vs baseline: 1.1838x; 1.1838x over previous
import functools

import jax
import jax.numpy as jnp
from jax import lax
from jax.experimental import pallas as pl
from jax.experimental.pallas import tpu as pltpu
from jax.experimental.pallas import tpu_sc as plsc

N_NODES = 100000
N_EDGES = 3200000
D_FEAT = 128
D_EDGE = 16
N_GRAPHS = 64
D_U = 64
OUT_DIM = 128

NC, NS, L = 2, 16, 16
NW = NC * NS
CE = 1024
NCH = N_EDGES // CE
CH_PER_W = 2 * (-(-NCH // (2 * NW)))
BPK_WORDS = N_NODES // 4

BN = 2000
NB = N_NODES // BN


def _sc_edge_body(eidx_hbm, eattr_hbm, bpk_hbm, out_hbm,
                  bpk_v, idx_v0, idx_v1, attr_v0, attr_v1, acc_v,
                  isem0, isem1, asem0, asem1):
    c = lax.axis_index("c")
    s = lax.axis_index("s")
    wid = s * NC + c
    idx_b = (idx_v0, idx_v1)
    attr_b = (attr_v0, attr_v1)
    isem = (isem0, isem1)
    asem = (asem0, asem1)

    pltpu.sync_copy(bpk_hbm, bpk_v)

    def zrow(r, carry):
        acc_v[pl.ds(r * L, L)] = jnp.zeros((L,), jnp.float32)
        return carry
    lax.fori_loop(0, N_GRAPHS, zrow, 0)

    lane = lax.iota(jnp.int32, L)

    def issue(jj, b):
        cid = wid + jj * NW

        @pl.when(cid < NCH)
        def _():
            base = cid * CE
            pltpu.make_async_copy(
                eidx_hbm.at[0, pl.ds(base, CE)], idx_b[b], isem[b]).start()
            pltpu.make_async_copy(
                eattr_hbm.at[pl.ds(base * D_EDGE, CE * D_EDGE)],
                attr_b[b], asem[b]).start()

    def consume(jj, b):
        cid = wid + jj * NW

        @pl.when(cid < NCH)
        def _():
            base = cid * CE
            pltpu.make_async_copy(
                eidx_hbm.at[0, pl.ds(base, CE)], idx_b[b], isem[b]).wait()
            pltpu.make_async_copy(
                eattr_hbm.at[pl.ds(base * D_EDGE, CE * D_EDGE)],
                attr_b[b], asem[b]).wait()

            def group(e16, carry2):
                idx16 = idx_b[b][pl.ds(e16 * L, L)]
                w = plsc.load_gather(
                    bpk_v, [lax.shift_right_logical(idx16, 2)])
                seg16 = lax.shift_right_logical(w, (idx16 & 3) * 8) & 255
                addr16 = seg16 * D_EDGE
                for k in range(L):
                    row = attr_b[b][pl.ds((e16 * L + k) * D_EDGE, D_EDGE)]
                    a = jnp.take(addr16, jnp.full((L,), k, jnp.int32))
                    plsc.addupdate_scatter(acc_v, [a + lane], row)
                return carry2
            lax.fori_loop(0, CE // L, group, 0)

    issue(0, 0)

    def outer(j2, carry):
        jj = j2 * 2
        issue(jj + 1, 1)
        consume(jj, 0)
        issue(jj + 2, 0)
        consume(jj + 1, 1)
        return carry
    lax.fori_loop(0, CH_PER_W // 2, outer, 0)

    pltpu.sync_copy(acc_v, out_hbm.at[wid])


_sc_edge = functools.partial(
    pl.kernel,
    out_type=jax.ShapeDtypeStruct((NW, N_GRAPHS * D_EDGE), jnp.float32),
    mesh=plsc.VectorSubcoreMesh(core_axis_name="c", subcore_axis_name="s"),
    compiler_params=pltpu.CompilerParams(needs_layout_passes=False),
    scratch_types=[
        pltpu.VMEM((BPK_WORDS,), jnp.int32),
        pltpu.VMEM((CE,), jnp.int32),
        pltpu.VMEM((CE,), jnp.int32),
        pltpu.VMEM((CE * D_EDGE,), jnp.float32),
        pltpu.VMEM((CE * D_EDGE,), jnp.float32),
        pltpu.VMEM((N_GRAPHS * D_EDGE,), jnp.float32),
        pltpu.SemaphoreType.DMA,
        pltpu.SemaphoreType.DMA,
        pltpu.SemaphoreType.DMA,
        pltpu.SemaphoreType.DMA,
    ],
)(_sc_edge_body)


def _tc_body(batch_ref, x_ref, u_ref, W_ref, b_ref, ep_ref,
             out_ref, acc_node):
    t = pl.program_id(0)

    @pl.when(t == 0)
    def _init():
        acc_node[...] = jnp.zeros_like(acc_node)

    @pl.when(t < NB)
    def _node():
        b_blk = batch_ref[0, 0, :].astype(jnp.int32)[None, :]
        g = lax.broadcasted_iota(jnp.int32, (N_GRAPHS, 1), 0)
        onehot_t = (b_blk == g).astype(jnp.float32)
        acc_node[...] += jnp.dot(onehot_t, x_ref[...],
                                 preferred_element_type=jnp.float32)

    @pl.when(t == NB)
    def _final():
        edge_agg = jnp.sum(ep_ref[...], axis=0)
        inp = jnp.concatenate(
            [acc_node[...], edge_agg, u_ref[...]], axis=-1)
        out = jnp.dot(inp, W_ref[...], preferred_element_type=jnp.float32)
        out_ref[...] = jnp.maximum(out + b_ref[...], 0.0)


@jax.jit
def kernel(x, edge_index, edge_attr, u, batch, W, b):
    batch_i32 = batch.astype(jnp.int32)
    eidx = edge_index.astype(jnp.int32)
    eattr_flat = edge_attr.reshape(-1)

    bpk = lax.bitcast_convert_type(
        batch_i32.astype(jnp.uint8).reshape(BPK_WORDS, 4), jnp.int32)
    edge_part = _sc_edge(eidx, eattr_flat, bpk)
    edge_part = edge_part.reshape(NW, N_GRAPHS, D_EDGE)

    batch3 = batch_i32.reshape(NB, 1, BN)
    b2 = b.reshape(1, OUT_DIM)
    out = pl.pallas_call(
        _tc_body,
        grid=(NB + 1,),
        in_specs=[
            pl.BlockSpec((1, 1, BN), lambda t: (jnp.minimum(t, NB - 1), 0, 0)),
            pl.BlockSpec((BN, D_FEAT), lambda t: (jnp.minimum(t, NB - 1), 0)),
            pl.BlockSpec((N_GRAPHS, D_U), lambda t: (0, 0)),
            pl.BlockSpec((D_FEAT + D_EDGE + D_U, OUT_DIM), lambda t: (0, 0)),
            pl.BlockSpec((1, OUT_DIM), lambda t: (0, 0)),
            pl.BlockSpec((NW, N_GRAPHS, D_EDGE), lambda t: (0, 0, 0)),
        ],
        out_specs=pl.BlockSpec((N_GRAPHS, OUT_DIM), lambda t: (0, 0)),
        out_shape=jax.ShapeDtypeStruct((N_GRAPHS, OUT_DIM), jnp.float32),
        scratch_shapes=[
            pltpu.VMEM((N_GRAPHS, D_FEAT), jnp.float32),
        ],
    )(batch3, x, u, W, b2, edge_part)
    return out
